# baseline (speedup 1.0000x reference)
import functools

import jax
import jax.numpy as jnp
from jax import lax
from jax.experimental import pallas as pl
from jax.experimental.pallas import tpu as pltpu
from jax.experimental.pallas import tpu_sc as plsc

N = 2048
D = 768
H = 1536
E = 64
TM = 128
G = 96
GR = G * TM
NW = 32
CP = (2 * N) // NW
CT = N // NW
CHUNK = 256
CPH = CP // 2


def _meta_body(x_ref, gw_ref, l128_ref, l32_ref, u64_ref,
               rows_ref, pw_ref, te_ref):
    xx = x_ref[...]
    logits = jnp.dot(xx, gw_ref[...], preferred_element_type=jnp.float32)
    ii = lax.broadcasted_iota(jnp.int32, (N, E), 1)
    big = jnp.int32(1 << 30)
    m1 = jnp.max(logits, axis=1, keepdims=True)
    a1 = jnp.min(jnp.where(logits == m1, ii, big), axis=1, keepdims=True)
    l2 = jnp.where(ii == a1, jnp.float32(-1e30), logits)
    m2 = jnp.max(l2, axis=1, keepdims=True)
    a2 = jnp.min(jnp.where(l2 == m2, ii, big), axis=1, keepdims=True)
    r = jnp.exp(m2 - m1)
    w_top1 = 1.0 / (1.0 + r)
    w_top2 = r / (1.0 + r)
    e1 = (ii == a1).astype(jnp.float32)
    e2 = (ii == a2).astype(jnp.float32)
    onehot = jnp.concatenate([e1, e2], axis=0)
    l128 = l128_ref[...]
    nchunk = (2 * N) // CHUNK
    parts = []
    sums = []
    for c in range(nchunk):
        chunk = onehot[c * CHUNK:(c + 1) * CHUNK]
        parts.append(jnp.dot(l128, chunk, preferred_element_type=jnp.float32))
        sums.append(jnp.sum(chunk, axis=0, keepdims=True))
    s = jnp.concatenate(sums, axis=0)
    p = jnp.dot(l32_ref[...], s, preferred_element_type=jnp.float32)
    rank = jnp.concatenate(
        [parts[c] + p[c:c + 1] for c in range(nchunk)], axis=0)
    cnt = jnp.sum(s, axis=0, keepdims=True)
    cpad = jnp.floor((cnt + (TM - 1.0)) * (1.0 / TM)) * TM
    off = jnp.dot(cpad, u64_ref[...], preferred_element_type=jnp.float32)
    rank_p = jnp.sum(rank * onehot, axis=1, keepdims=True)
    off_p = jnp.sum(off * onehot, axis=1, keepdims=True)
    rows_ref[...] = (rank_p + off_p).astype(jnp.int32)
    pw_ref[...] = jnp.concatenate([w_top1, w_top2], axis=0)
    ti = lax.broadcasted_iota(jnp.int32, (G, E), 0).astype(jnp.float32)
    offt = off * (1.0 / TM)
    nt = cpad * (1.0 / TM)
    ind = jnp.where((ti >= offt) & (ti < offt + nt), 1.0, 0.0)
    ee = lax.broadcasted_iota(jnp.int32, (G, E), 1).astype(jnp.float32)
    has = jnp.where(cpad > 0.0, 1.0, 0.0)
    emax = jnp.max(ee[0:1] * has, axis=1, keepdims=True)
    used = jnp.sum(nt, axis=1, keepdims=True)
    anyt = jnp.sum(ind, axis=1, keepdims=True)
    teval = jnp.sum(ind * ee, axis=1, keepdims=True) + (1.0 - anyt) * emax
    te_ref[0:G] = teval.astype(jnp.int32)
    te_ref[G:G + 8] = jnp.broadcast_to(used, (8, 1)).astype(jnp.int32)


def _run_meta(x2, gate_W, interpret=False):
    l128 = jnp.tril(jnp.ones((CHUNK, CHUNK), jnp.float32), -1)
    l32 = jnp.tril(
        jnp.ones(((2 * N) // CHUNK, (2 * N) // CHUNK), jnp.float32), -1)
    u64 = jnp.triu(jnp.ones((E, E), jnp.float32), 1)
    rows, pw, te = pl.pallas_call(
        _meta_body,
        out_shape=[
            jax.ShapeDtypeStruct((2 * N, 1), jnp.int32),
            jax.ShapeDtypeStruct((2 * N, 1), jnp.float32),
            jax.ShapeDtypeStruct((G + 8, 1), jnp.int32),
        ],
        interpret=interpret,
    )(x2, gate_W, l128, l32, u64)
    return rows.reshape(2 * N), pw.reshape(2 * N), te.reshape(G + 8)


def _gmm_body(te_ref, a_ref, w1_ref, b1_ref, w2_ref, b2_ref, y_ref):
    @pl.when(pl.program_id(0) < te_ref[G])
    def _():
        a = a_ref[...]
        h = (jnp.dot(a, w1_ref[0], preferred_element_type=jnp.float32,
                     precision=lax.Precision.DEFAULT)
             + b1_ref[0])
        h = 0.5 * h * (1.0 + lax.erf(h * 0.7071067811865476))
        y_ref[...] = (
            jnp.dot(h, w2_ref[0], preferred_element_type=jnp.float32,
                    precision=lax.Precision.DEFAULT)
            + b2_ref[0])


def _run_gmm(te, a, w1, b1, w2, b2, interpret=False):
    grid_spec = pltpu.PrefetchScalarGridSpec(
        num_scalar_prefetch=1,
        grid=(G,),
        in_specs=[
            pl.BlockSpec((TM, D), lambda i, te: (jnp.minimum(i, te[G] - 1), 0)),
            pl.BlockSpec((1, D, H), lambda i, te: (te[i], 0, 0)),
            pl.BlockSpec((1, 1, H), lambda i, te: (te[i], 0, 0)),
            pl.BlockSpec((1, H, D), lambda i, te: (te[i], 0, 0)),
            pl.BlockSpec((1, 1, D), lambda i, te: (te[i], 0, 0)),
        ],
        out_specs=pl.BlockSpec(
            (TM, D), lambda i, te: (jnp.minimum(i, te[G] - 1), 0)),
    )
    return pl.pallas_call(
        _gmm_body,
        grid_spec=grid_spec,
        out_shape=jax.ShapeDtypeStruct((GR, D), jnp.float32),
        interpret=interpret,
    )(te, a, w1, b1, w2, b2)


def _make_dispatch():
    mesh = plsc.VectorSubcoreMesh(core_axis_name="c", subcore_axis_name="s")

    @functools.partial(
        pl.kernel,
        mesh=mesh,
        out_type=jax.ShapeDtypeStruct((GR, D), jnp.float32),
        scratch_types=[
            pltpu.VMEM((CP,), jnp.int32),
            pltpu.VMEM((CPH,), jnp.int32),
            pltpu.VMEM((CPH,), jnp.int32),
            pltpu.VMEM((CPH, D), jnp.float32),
            pltpu.VMEM((CPH, D), jnp.float32),
            pltpu.SemaphoreType.DMA,
            pltpu.SemaphoreType.DMA,
            pltpu.SemaphoreType.DMA,
            pltpu.SemaphoreType.DMA,
            pltpu.SemaphoreType.DMA,
            pltpu.SemaphoreType.DMA,
            pltpu.SemaphoreType.DMA,
        ],
    )
    def dispatch(x_hbm, tok_hbm, rows_hbm, a_hbm,
                 tok_v, rows_v0, rows_v1, buf_a, buf_b,
                 st, sr0, sr1, g0, g1, w0s, w1s):
        wid = lax.axis_index("s") * 2 + lax.axis_index("c")
        base = wid * CP
        ct = pltpu.async_copy(tok_hbm.at[pl.ds(base, CP)], tok_v, st)
        cr0 = pltpu.async_copy(rows_hbm.at[pl.ds(base, CPH)], rows_v0, sr0)
        cr1 = pltpu.async_copy(
            rows_hbm.at[pl.ds(base + CPH, CPH)], rows_v1, sr1)
        ct.wait()
        cg0 = pltpu.async_copy(x_hbm.at[tok_v.at[pl.ds(0, CPH)]], buf_a, g0)
        cg1 = pltpu.async_copy(x_hbm.at[tok_v.at[pl.ds(CPH, CPH)]], buf_b, g1)
        cg0.wait()
        cr0.wait()
        cs0 = pltpu.async_copy(buf_a, a_hbm.at[rows_v0], w0s)
        cg1.wait()
        cr1.wait()
        cs1 = pltpu.async_copy(buf_b, a_hbm.at[rows_v1], w1s)
        cs0.wait()
        cs1.wait()

    return dispatch


def _make_combine():
    mesh = plsc.VectorSubcoreMesh(core_axis_name="c", subcore_axis_name="s")

    @functools.partial(
        pl.kernel,
        mesh=mesh,
        out_type=jax.ShapeDtypeStruct((N, D), jnp.float32),
        scratch_types=[
            pltpu.VMEM((CT,), jnp.int32),
            pltpu.VMEM((CT,), jnp.int32),
            pltpu.VMEM((CT, 16), jnp.float32),
            pltpu.VMEM((CT, 16), jnp.float32),
            pltpu.VMEM((CT, D), jnp.float32),
            pltpu.VMEM((CT, D), jnp.float32),
            pltpu.SemaphoreType.DMA,
            pltpu.SemaphoreType.DMA,
            pltpu.SemaphoreType.DMA,
            pltpu.SemaphoreType.DMA,
            pltpu.SemaphoreType.DMA,
            pltpu.SemaphoreType.DMA,
        ],
    )
    def combine(y_hbm, rows_hbm, pwrep_hbm, out_hbm,
                idx0, idx1, pw0, pw1, buf0, buf1,
                s0, s1, si0, si1, sp0, sp1):
        wid = lax.axis_index("s") * 2 + lax.axis_index("c")
        base = wid * CT
        ci0 = pltpu.async_copy(rows_hbm.at[pl.ds(base, CT)], idx0, si0)
        ci1 = pltpu.async_copy(rows_hbm.at[pl.ds(N + base, CT)], idx1, si1)
        cp0 = pltpu.async_copy(pwrep_hbm.at[pl.ds(base, CT)], pw0, sp0)
        cp1 = pltpu.async_copy(pwrep_hbm.at[pl.ds(N + base, CT)], pw1, sp1)
        ci0.wait()
        ci1.wait()
        cg0 = pltpu.async_copy(y_hbm.at[idx0], buf0, s0)
        cg1 = pltpu.async_copy(y_hbm.at[idx1], buf1, s1)
        cp0.wait()
        cp1.wait()
        cg0.wait()
        cg1.wait()

        def tok_body(i, carry):
            w0 = pw0[i, pl.ds(0, 16)]
            w1v = pw1[i, pl.ds(0, 16)]
            for j in range(D // 16):
                sl = pl.ds(j * 16, 16)
                buf0[i, sl] = buf0[i, sl] * w0 + buf1[i, sl] * w1v
            return carry

        lax.fori_loop(0, CT, tok_body, 0)
        pltpu.sync_copy(buf0, out_hbm.at[pl.ds(base, CT)])

    return combine


_make_dispatch = functools.cache(_make_dispatch)
_make_combine = functools.cache(_make_combine)


def kernel(x, gate_W, w1, b1, w2, b2):
    x2 = x.reshape(N, D)
    rows, pw, te = _run_meta(x2, gate_W)
    tok = jnp.concatenate([jnp.arange(N, dtype=jnp.int32)] * 2)
    pwrep = jnp.broadcast_to(pw[:, None], (2 * N, 16))
    a = _make_dispatch()(x2, tok, rows)
    y = _run_gmm(te, a, w1, b1, w2, b2)
    out = _make_combine()(y, rows, pwrep)
    return out.reshape(1, N, D)


# speedup vs baseline: 1.0234x; 1.0234x over previous
import functools

import jax
import jax.numpy as jnp
from jax import lax
from jax.experimental import pallas as pl
from jax.experimental.pallas import tpu as pltpu
from jax.experimental.pallas import tpu_sc as plsc

N = 2048
D = 768
H = 1536
E = 64
TM = 128
G = 96
GR = G * TM
NW = 32
CP = (2 * N) // NW
CT = N // NW
CHUNK = 256
CPH = CP // 2


def _tril_strict(n):
    r = lax.broadcasted_iota(jnp.int32, (n, n), 0)
    c = lax.broadcasted_iota(jnp.int32, (n, n), 1)
    return jnp.where(r > c, 1.0, 0.0).astype(jnp.float32)


def _triu_strict(n):
    r = lax.broadcasted_iota(jnp.int32, (n, n), 0)
    c = lax.broadcasted_iota(jnp.int32, (n, n), 1)
    return jnp.where(r < c, 1.0, 0.0).astype(jnp.float32)


def _meta_body(x_ref, gw_ref, rows_ref, pw_ref, te_ref):
    xx = x_ref[...]
    logits = jnp.dot(xx, gw_ref[...], preferred_element_type=jnp.float32)
    ii = lax.broadcasted_iota(jnp.int32, (N, E), 1)
    big = jnp.int32(1 << 30)
    m1 = jnp.max(logits, axis=1, keepdims=True)
    a1 = jnp.min(jnp.where(logits == m1, ii, big), axis=1, keepdims=True)
    l2 = jnp.where(ii == a1, jnp.float32(-1e30), logits)
    m2 = jnp.max(l2, axis=1, keepdims=True)
    a2 = jnp.min(jnp.where(l2 == m2, ii, big), axis=1, keepdims=True)
    r = jnp.exp(m2 - m1)
    w_top1 = 1.0 / (1.0 + r)
    w_top2 = r / (1.0 + r)
    e1 = (ii == a1).astype(jnp.float32)
    e2 = (ii == a2).astype(jnp.float32)
    onehot = jnp.concatenate([e1, e2], axis=0)
    l128 = _tril_strict(CHUNK)
    nchunk = (2 * N) // CHUNK
    parts = []
    sums = []
    for c in range(nchunk):
        chunk = onehot[c * CHUNK:(c + 1) * CHUNK]
        parts.append(jnp.dot(l128, chunk, preferred_element_type=jnp.float32))
        sums.append(jnp.sum(chunk, axis=0, keepdims=True))
    s = jnp.concatenate(sums, axis=0)
    p = jnp.dot(_tril_strict(nchunk), s, preferred_element_type=jnp.float32)
    rank = jnp.concatenate(
        [parts[c] + p[c:c + 1] for c in range(nchunk)], axis=0)
    cnt = jnp.sum(s, axis=0, keepdims=True)
    cpad = jnp.floor((cnt + (TM - 1.0)) * (1.0 / TM)) * TM
    off = jnp.dot(cpad, _triu_strict(E), preferred_element_type=jnp.float32)
    rank_p = jnp.sum(rank * onehot, axis=1, keepdims=True)
    off_p = jnp.sum(off * onehot, axis=1, keepdims=True)
    rows_ref[...] = (rank_p + off_p).astype(jnp.int32)
    pw_ref[...] = jnp.concatenate([w_top1, w_top2], axis=0)
    ti = lax.broadcasted_iota(jnp.int32, (G, E), 0).astype(jnp.float32)
    offt = off * (1.0 / TM)
    nt = cpad * (1.0 / TM)
    ind = jnp.where((ti >= offt) & (ti < offt + nt), 1.0, 0.0)
    ee = lax.broadcasted_iota(jnp.int32, (G, E), 1).astype(jnp.float32)
    has = jnp.where(cpad > 0.0, 1.0, 0.0)
    emax = jnp.max(ee[0:1] * has, axis=1, keepdims=True)
    used = jnp.sum(nt, axis=1, keepdims=True)
    anyt = jnp.sum(ind, axis=1, keepdims=True)
    teval = jnp.sum(ind * ee, axis=1, keepdims=True) + (1.0 - anyt) * emax
    te_ref[0:G] = teval.astype(jnp.int32)
    te_ref[G:G + 8] = jnp.broadcast_to(used, (8, 1)).astype(jnp.int32)


def _run_meta(x2, gate_W, interpret=False):
    rows, pw, te = pl.pallas_call(
        _meta_body,
        out_shape=[
            jax.ShapeDtypeStruct((2 * N, 1), jnp.int32),
            jax.ShapeDtypeStruct((2 * N, 1), jnp.float32),
            jax.ShapeDtypeStruct((G + 8, 1), jnp.int32),
        ],
        interpret=interpret,
    )(x2, gate_W)
    return rows.reshape(2 * N), pw.reshape(2 * N), te.reshape(G + 8)


def _gmm_body(te_ref, a_ref, w1_ref, b1_ref, w2_ref, b2_ref, y_ref):
    @pl.when(pl.program_id(0) < te_ref[G])
    def _():
        a = a_ref[...]
        h = (jnp.dot(a, w1_ref[0], preferred_element_type=jnp.float32,
                     precision=lax.Precision.DEFAULT)
             + b1_ref[0])
        h = 0.5 * h * (1.0 + lax.erf(h * 0.7071067811865476))
        y_ref[...] = (
            jnp.dot(h, w2_ref[0], preferred_element_type=jnp.float32,
                    precision=lax.Precision.DEFAULT)
            + b2_ref[0])


def _run_gmm(te, a, w1, b1, w2, b2, interpret=False):
    grid_spec = pltpu.PrefetchScalarGridSpec(
        num_scalar_prefetch=1,
        grid=(G,),
        in_specs=[
            pl.BlockSpec((TM, D), lambda i, te: (jnp.minimum(i, te[G] - 1), 0)),
            pl.BlockSpec((1, D, H), lambda i, te: (te[i], 0, 0)),
            pl.BlockSpec((1, 1, H), lambda i, te: (te[i], 0, 0)),
            pl.BlockSpec((1, H, D), lambda i, te: (te[i], 0, 0)),
            pl.BlockSpec((1, 1, D), lambda i, te: (te[i], 0, 0)),
        ],
        out_specs=pl.BlockSpec(
            (TM, D), lambda i, te: (jnp.minimum(i, te[G] - 1), 0)),
    )
    return pl.pallas_call(
        _gmm_body,
        grid_spec=grid_spec,
        out_shape=jax.ShapeDtypeStruct((GR, D), jnp.float32),
        interpret=interpret,
    )(te, a, w1, b1, w2, b2)


def _make_dispatch():
    mesh = plsc.VectorSubcoreMesh(core_axis_name="c", subcore_axis_name="s")

    @functools.partial(
        pl.kernel,
        mesh=mesh,
        out_type=jax.ShapeDtypeStruct((GR, D), jnp.float32),
        scratch_types=[
            pltpu.VMEM((CP,), jnp.int32),
            pltpu.VMEM((CPH,), jnp.int32),
            pltpu.VMEM((CPH,), jnp.int32),
            pltpu.VMEM((CPH, D), jnp.float32),
            pltpu.VMEM((CPH, D), jnp.float32),
            pltpu.SemaphoreType.DMA,
            pltpu.SemaphoreType.DMA,
            pltpu.SemaphoreType.DMA,
            pltpu.SemaphoreType.DMA,
            pltpu.SemaphoreType.DMA,
            pltpu.SemaphoreType.DMA,
            pltpu.SemaphoreType.DMA,
        ],
    )
    def dispatch(x_hbm, tok_hbm, rows_hbm, a_hbm,
                 tok_v, rows_v0, rows_v1, buf_a, buf_b,
                 st, sr0, sr1, g0, g1, w0s, w1s):
        wid = lax.axis_index("s") * 2 + lax.axis_index("c")
        base = wid * CP
        ct = pltpu.async_copy(tok_hbm.at[pl.ds(base, CP)], tok_v, st)
        cr0 = pltpu.async_copy(rows_hbm.at[pl.ds(base, CPH)], rows_v0, sr0)
        cr1 = pltpu.async_copy(
            rows_hbm.at[pl.ds(base + CPH, CPH)], rows_v1, sr1)
        ct.wait()
        cg0 = pltpu.async_copy(x_hbm.at[tok_v.at[pl.ds(0, CPH)]], buf_a, g0)
        cg1 = pltpu.async_copy(x_hbm.at[tok_v.at[pl.ds(CPH, CPH)]], buf_b, g1)
        cg0.wait()
        cr0.wait()
        cs0 = pltpu.async_copy(buf_a, a_hbm.at[rows_v0], w0s)
        cg1.wait()
        cr1.wait()
        cs1 = pltpu.async_copy(buf_b, a_hbm.at[rows_v1], w1s)
        cs0.wait()
        cs1.wait()

    return dispatch


def _make_combine():
    mesh = plsc.VectorSubcoreMesh(core_axis_name="c", subcore_axis_name="s")

    @functools.partial(
        pl.kernel,
        mesh=mesh,
        out_type=jax.ShapeDtypeStruct((N, D), jnp.float32),
        scratch_types=[
            pltpu.VMEM((CT,), jnp.int32),
            pltpu.VMEM((CT,), jnp.int32),
            pltpu.VMEM((CT, 16), jnp.float32),
            pltpu.VMEM((CT, 16), jnp.float32),
            pltpu.VMEM((CT, D), jnp.float32),
            pltpu.VMEM((CT, D), jnp.float32),
            pltpu.SemaphoreType.DMA,
            pltpu.SemaphoreType.DMA,
            pltpu.SemaphoreType.DMA,
            pltpu.SemaphoreType.DMA,
            pltpu.SemaphoreType.DMA,
            pltpu.SemaphoreType.DMA,
        ],
    )
    def combine(y_hbm, rows_hbm, pwrep_hbm, out_hbm,
                idx0, idx1, pw0, pw1, buf0, buf1,
                s0, s1, si0, si1, sp0, sp1):
        wid = lax.axis_index("s") * 2 + lax.axis_index("c")
        base = wid * CT
        ci0 = pltpu.async_copy(rows_hbm.at[pl.ds(base, CT)], idx0, si0)
        ci1 = pltpu.async_copy(rows_hbm.at[pl.ds(N + base, CT)], idx1, si1)
        cp0 = pltpu.async_copy(pwrep_hbm.at[pl.ds(base, CT)], pw0, sp0)
        cp1 = pltpu.async_copy(pwrep_hbm.at[pl.ds(N + base, CT)], pw1, sp1)
        ci0.wait()
        ci1.wait()
        cg0 = pltpu.async_copy(y_hbm.at[idx0], buf0, s0)
        cg1 = pltpu.async_copy(y_hbm.at[idx1], buf1, s1)
        cp0.wait()
        cp1.wait()
        cg0.wait()
        cg1.wait()

        def tok_body(i, carry):
            w0 = pw0[i, pl.ds(0, 16)]
            w1v = pw1[i, pl.ds(0, 16)]
            for j in range(D // 16):
                sl = pl.ds(j * 16, 16)
                buf0[i, sl] = buf0[i, sl] * w0 + buf1[i, sl] * w1v
            return carry

        lax.fori_loop(0, CT, tok_body, 0)
        pltpu.sync_copy(buf0, out_hbm.at[pl.ds(base, CT)])

    return combine


_make_dispatch = functools.cache(_make_dispatch)
_make_combine = functools.cache(_make_combine)


def kernel(x, gate_W, w1, b1, w2, b2):
    x2 = x.reshape(N, D)
    rows, pw, te = _run_meta(x2, gate_W)
    tok = jnp.concatenate([jnp.arange(N, dtype=jnp.int32)] * 2)
    pwrep = jnp.broadcast_to(pw[:, None], (2 * N, 16))
    a = _make_dispatch()(x2, tok, rows)
    y = _run_gmm(te, a, w1, b1, w2, b2)
    out = _make_combine()(y, rows, pwrep)
    return out.reshape(1, N, D)
